# baseline (speedup 1.0000x reference)
import functools

import jax
import jax.numpy as jnp
import numpy as np
from jax import lax
from jax.experimental import pallas as pl
from jax.experimental.pallas import tpu as pltpu
from jax.experimental.pallas import tpu_sc as plsc

_PCM = np.array([
    [1, 0, 1, 0, 1, 0, 1, 0, 1, 0, 1, 0, 1, 0, 1, 0, 1, 0, 1, 0, 1, 0, 1, 0, 1, 0, 1, 0, 1, 0, 1],
    [0, 1, 1, 0, 0, 1, 1, 0, 0, 1, 1, 0, 0, 1, 1, 0, 0, 1, 1, 0, 0, 1, 1, 0, 0, 1, 1, 0, 0, 1, 1],
    [0, 0, 0, 1, 1, 1, 1, 0, 0, 0, 0, 1, 1, 1, 1, 0, 0, 0, 0, 1, 1, 1, 1, 0, 0, 0, 0, 1, 1, 1, 1],
    [0, 0, 0, 0, 0, 0, 0, 1, 1, 1, 1, 1, 1, 1, 1, 0, 0, 0, 0, 0, 0, 0, 0, 1, 1, 1, 1, 1, 1, 1, 1],
    [0, 0, 0, 0, 0, 0, 0, 0, 0, 0, 0, 0, 0, 0, 0, 1, 1, 1, 1, 1, 1, 1, 1, 1, 1, 1, 1, 1, 1, 1, 1],
], dtype=np.int64)
_ROLLED = np.stack(np.where(_PCM), axis=1)
_NCHK, _NVAR = _PCM.shape
_E = _ROLLED.shape[0]
_DEG = 16
_COLS = _ROLLED[:, 1].reshape(_NCHK, _DEG)
_COL_EDGES = [np.where(_ROLLED[:, 1] == v)[0].tolist() for v in range(_NVAR)]
_NUM_ITER = 20

_SQRT2 = 1.4142135
_LN2_HI = 0.69314575
_LN2_LO = 1.4286068e-06
_INVLN2 = 1.4426950408889634
_LOGC = [-6.900793061981325e-10, 0.9999999966211686, -0.4999996532905042,
         0.3333335964540943, -0.25002812462237173, 0.2000147231976227,
         -0.16586843259976403, 0.14176370184103376, -0.13388417444788703,
         0.12990627398860688, -0.07417228391986355]
_EXPC = [0.9999999999595321, 1.000000037739721, 0.5000000107781664,
         0.16666415422747397, 0.04166621818498068, 0.008375133426734613,
         0.0013948586767683234]
_RCPC = [2.8499173034659373, -2.9844526530961426, 1.3616308933192882,
         -0.22857251654217783]


def _horner(coefs, x):
    acc = jnp.full(x.shape, coefs[-1], x.dtype)
    for c in coefs[-2::-1]:
        acc = acc * x + c
    return acc


def _sc_log(e):
    i = plsc.bitcast(e, jnp.int32)
    k = (i >> 23) - 127
    m = plsc.bitcast((i & 0x7FFFFF) | 0x3F800000, jnp.float32)
    big = m > _SQRT2
    m = jnp.where(big, m * 0.5, m)
    kf = (k + jnp.where(big, 1, 0)).astype(jnp.float32)
    p = _horner(_LOGC, m - 1.0)
    return kf * _LN2_HI + (kf * _LN2_LO + p)


def _sc_expneg(a):
    a = jnp.minimum(a, 18.0)
    n = (a * _INVLN2 + 0.5).astype(jnp.int32)
    nf = n.astype(jnp.float32)
    f = (a - nf * _LN2_HI) - nf * _LN2_LO
    ef = _horner(_EXPC, -f)
    return ef * plsc.bitcast((127 - n) << 23, jnp.float32)


def _sc_rcp1to2(u):
    r = _horner(_RCPC, u)
    r = r * (2.0 - u * r)
    return r * (2.0 - u * r)


def _sc_tanh12(y):
    t = _sc_expneg(jnp.abs(y))
    q = (1.0 - t) * _sc_rcp1to2(1.0 + t)
    return jnp.where(y < 0.0, -q, q)


def _loo_products(grp):
    n = len(grp)
    pref = [grp[0]]
    for k in range(1, n):
        pref.append(pref[-1] * grp[k])
    suf = [grp[n - 1]]
    for k in range(n - 2, -1, -1):
        suf.append(suf[-1] * grp[k])
    suf = suf[::-1]
    out = []
    for k in range(n):
        if k == 0:
            out.append(suf[1])
        elif k == n - 1:
            out.append(pref[n - 2])
        else:
            out.append(pref[k - 1] * suf[k + 1])
    return out


def _sc_bp_body(pb, llr_hbm, out_hbm, llr_v, msg_v, he_v, out_v):
    wid = lax.axis_index("s") * 2 + lax.axis_index("c")
    pltpu.sync_copy(llr_hbm.at[wid], llr_v)

    def g_body(g, carry):
        lanes = pl.ds(g * 16, 16)
        for v in range(_NVAR):
            t = _sc_tanh12(llr_v[v, lanes])
            for e in _COL_EDGES[v]:
                msg_v[e, lanes] = t

        def it_body(it, c2):
            cs = [None] * _NVAR
            for c in range(_NCHK):
                grp = [msg_v[c * _DEG + k, lanes] for k in range(_DEG)]
                loo = _loo_products(grp)
                for k in range(_DEG):
                    e1 = jnp.clip(1.0 + loo[k], 1e-07, 2.0 - 1e-07)
                    e2 = jnp.clip(1.0 - loo[k], 1e-07, 2.0 - 1e-07)
                    he = _sc_log(e1) - _sc_log(e2)
                    e = c * _DEG + k
                    he_v[e, lanes] = he
                    v = int(_COLS[c, k])
                    cs[v] = he if cs[v] is None else cs[v] + he
            for v in range(_NVAR):
                base = cs[v] + llr_v[v, lanes]
                out_v[v, lanes] = base
                for e in _COL_EDGES[v]:
                    msg_v[e, lanes] = _sc_tanh12(base - he_v[e, lanes])
            return c2

        lax.fori_loop(0, _NUM_ITER, it_body, 0)
        return carry

    lax.fori_loop(0, pb // 16, g_body, 0)
    pltpu.sync_copy(out_v, out_hbm.at[wid])


_NW = 32


@functools.partial(jax.jit, static_argnames=("pb",))
def _sc_bp(llr_sc, pb):
    mesh = plsc.VectorSubcoreMesh(
        core_axis_name="c", subcore_axis_name="s", num_cores=2, num_subcores=16)
    return pl.kernel(
        functools.partial(_sc_bp_body, pb),
        out_type=jax.ShapeDtypeStruct((_NW, _NVAR, pb), jnp.float32),
        mesh=mesh,
        compiler_params=pltpu.CompilerParams(needs_layout_passes=False),
        scratch_types=[
            pltpu.VMEM((_NVAR, pb), jnp.float32),
            pltpu.VMEM((_E, pb), jnp.float32),
            pltpu.VMEM((_E, pb), jnp.float32),
            pltpu.VMEM((_NVAR, pb), jnp.float32),
        ],
    )(llr_sc)


@jax.jit
def kernel(llr):
    B = llr.shape[0]
    pb = B // _NW
    llr_sc = llr.T.reshape(_NVAR, _NW, pb).transpose(1, 0, 2)
    out_sc = _sc_bp(llr_sc, pb)
    return out_sc.transpose(1, 0, 2).reshape(_NVAR, B).T


# speedup vs baseline: 2.7354x; 2.7354x over previous
import functools

import jax
import jax.numpy as jnp
import numpy as np
from jax import lax
from jax.experimental import pallas as pl
from jax.experimental.pallas import tpu as pltpu
from jax.experimental.pallas import tpu_sc as plsc

_PCM = np.array([
    [1, 0, 1, 0, 1, 0, 1, 0, 1, 0, 1, 0, 1, 0, 1, 0, 1, 0, 1, 0, 1, 0, 1, 0, 1, 0, 1, 0, 1, 0, 1],
    [0, 1, 1, 0, 0, 1, 1, 0, 0, 1, 1, 0, 0, 1, 1, 0, 0, 1, 1, 0, 0, 1, 1, 0, 0, 1, 1, 0, 0, 1, 1],
    [0, 0, 0, 1, 1, 1, 1, 0, 0, 0, 0, 1, 1, 1, 1, 0, 0, 0, 0, 1, 1, 1, 1, 0, 0, 0, 0, 1, 1, 1, 1],
    [0, 0, 0, 0, 0, 0, 0, 1, 1, 1, 1, 1, 1, 1, 1, 0, 0, 0, 0, 0, 0, 0, 0, 1, 1, 1, 1, 1, 1, 1, 1],
    [0, 0, 0, 0, 0, 0, 0, 0, 0, 0, 0, 0, 0, 0, 0, 1, 1, 1, 1, 1, 1, 1, 1, 1, 1, 1, 1, 1, 1, 1, 1],
], dtype=np.int64)
_ROLLED = np.stack(np.where(_PCM), axis=1)
_NCHK, _NVAR = _PCM.shape
_E = _ROLLED.shape[0]
_DEG = 16
_COLS = _ROLLED[:, 1].reshape(_NCHK, _DEG)
_COL_EDGES = [np.where(_ROLLED[:, 1] == v)[0].tolist() for v in range(_NVAR)]
_NUM_ITER = 20

_SQRT2 = 1.4142135
_LN2_HI = 0.69314575
_LN2_LO = 1.4286068e-06
_INVLN2 = 1.4426950408889634
_LOGC = [-6.900793061981325e-10, 0.9999999966211686, -0.4999996532905042,
         0.3333335964540943, -0.25002812462237173, 0.2000147231976227,
         -0.16586843259976403, 0.14176370184103376, -0.13388417444788703,
         0.12990627398860688, -0.07417228391986355]
_EXPC = [0.9999999999595321, 1.000000037739721, 0.5000000107781664,
         0.16666415422747397, 0.04166621818498068, 0.008375133426734613,
         0.0013948586767683234]
_RCPC = [2.8499173034659373, -2.9844526530961426, 1.3616308933192882,
         -0.22857251654217783]


def _horner(coefs, x):
    acc = jnp.full(x.shape, coefs[-1], x.dtype)
    for c in coefs[-2::-1]:
        acc = acc * x + c
    return acc


_RLO = float(1e-7 / (2.0 - 1e-7))
_RHI = float((2.0 - 1e-7) / 1e-7)


def _sc_log(e):
    i = plsc.bitcast(e, jnp.int32)
    k = (i >> 23) - 127
    m = plsc.bitcast((i & 0x7FFFFF) | 0x3F800000, jnp.float32)
    big = m > _SQRT2
    m = jnp.where(big, m * 0.5, m)
    kf = (k + jnp.where(big, 1, 0)).astype(jnp.float32)
    p = _horner(_LOGC, m - 1.0)
    return kf * _LN2_HI + (kf * _LN2_LO + p)


def _sc_atanh2(x):
    r = jnp.clip((1.0 + x) / (1.0 - x), _RLO, _RHI)
    return _sc_log(r)


def _sc_tanh12(y):
    t = jnp.exp(-jnp.abs(y))
    q = (1.0 - t) / (1.0 + t)
    return jnp.where(y < 0.0, -q, q)


def _loo_products(grp):
    n = len(grp)
    pref = [grp[0]]
    for k in range(1, n):
        pref.append(pref[-1] * grp[k])
    suf = [grp[n - 1]]
    for k in range(n - 2, -1, -1):
        suf.append(suf[-1] * grp[k])
    suf = suf[::-1]
    out = []
    for k in range(n):
        if k == 0:
            out.append(suf[1])
        elif k == n - 1:
            out.append(pref[n - 2])
        else:
            out.append(pref[k - 1] * suf[k + 1])
    return out


def _sc_bp_body(pb, unroll, llr_hbm, out_hbm, llr_v, msg_v, he_v, out_v):
    wid = lax.axis_index("s") * 2 + lax.axis_index("c")
    pltpu.sync_copy(llr_hbm.at[wid], llr_v)
    ng = pb // 16

    @plsc.parallel_loop(0, ng, unroll=unroll)
    def _init(g):
        lanes = pl.ds(g * 16, 16)
        for v in range(_NVAR):
            t = _sc_tanh12(llr_v[v, lanes])
            for e in _COL_EDGES[v]:
                msg_v[e, lanes] = t

    def it_body(it, c2):
        @plsc.parallel_loop(0, ng, unroll=unroll)
        def _g(g):
            lanes = pl.ds(g * 16, 16)
            cs = [None] * _NVAR
            for c in range(_NCHK):
                grp = [msg_v[c * _DEG + k, lanes] for k in range(_DEG)]
                loo = _loo_products(grp)
                for k in range(_DEG):
                    he = _sc_atanh2(loo[k])
                    e = c * _DEG + k
                    he_v[e, lanes] = he
                    v = int(_COLS[c, k])
                    cs[v] = he if cs[v] is None else cs[v] + he
            for v in range(_NVAR):
                base = cs[v] + llr_v[v, lanes]
                out_v[v, lanes] = base
                for e in _COL_EDGES[v]:
                    msg_v[e, lanes] = _sc_tanh12(base - he_v[e, lanes])
        return c2

    lax.fori_loop(0, _NUM_ITER, it_body, 0)
    pltpu.sync_copy(out_v, out_hbm.at[wid])


_NW = 32


@functools.partial(jax.jit, static_argnames=("pb", "unroll"))
def _sc_bp(llr_sc, pb, unroll=2):
    mesh = plsc.VectorSubcoreMesh(
        core_axis_name="c", subcore_axis_name="s", num_cores=2, num_subcores=16)
    return pl.kernel(
        functools.partial(_sc_bp_body, pb, unroll),
        out_type=jax.ShapeDtypeStruct((_NW, _NVAR, pb), jnp.float32),
        mesh=mesh,
        compiler_params=pltpu.CompilerParams(needs_layout_passes=False),
        scratch_types=[
            pltpu.VMEM((_NVAR, pb), jnp.float32),
            pltpu.VMEM((_E, pb), jnp.float32),
            pltpu.VMEM((_E, pb), jnp.float32),
            pltpu.VMEM((_NVAR, pb), jnp.float32),
        ],
    )(llr_sc)


@jax.jit
def kernel(llr):
    B = llr.shape[0]
    pb = B // _NW
    llr_sc = llr.T.reshape(_NVAR, _NW, pb).transpose(1, 0, 2)
    out_sc = _sc_bp(llr_sc, pb)
    return out_sc.transpose(1, 0, 2).reshape(_NVAR, B).T


# speedup vs baseline: 75.0778x; 27.4470x over previous
import functools

import jax
import jax.numpy as jnp
import numpy as np
from jax import lax
from jax.experimental import pallas as pl
from jax.experimental.pallas import tpu as pltpu
from jax.experimental.pallas import tpu_sc as plsc

_PCM = np.array([
    [1, 0, 1, 0, 1, 0, 1, 0, 1, 0, 1, 0, 1, 0, 1, 0, 1, 0, 1, 0, 1, 0, 1, 0, 1, 0, 1, 0, 1, 0, 1],
    [0, 1, 1, 0, 0, 1, 1, 0, 0, 1, 1, 0, 0, 1, 1, 0, 0, 1, 1, 0, 0, 1, 1, 0, 0, 1, 1, 0, 0, 1, 1],
    [0, 0, 0, 1, 1, 1, 1, 0, 0, 0, 0, 1, 1, 1, 1, 0, 0, 0, 0, 1, 1, 1, 1, 0, 0, 0, 0, 1, 1, 1, 1],
    [0, 0, 0, 0, 0, 0, 0, 1, 1, 1, 1, 1, 1, 1, 1, 0, 0, 0, 0, 0, 0, 0, 0, 1, 1, 1, 1, 1, 1, 1, 1],
    [0, 0, 0, 0, 0, 0, 0, 0, 0, 0, 0, 0, 0, 0, 0, 1, 1, 1, 1, 1, 1, 1, 1, 1, 1, 1, 1, 1, 1, 1, 1],
], dtype=np.int64)
_ROLLED = np.stack(np.where(_PCM), axis=1)
_NCHK, _NVAR = _PCM.shape
_E = _ROLLED.shape[0]
_DEG = 16
_COLS = _ROLLED[:, 1].reshape(_NCHK, _DEG)
_COL_EDGES = [np.where(_ROLLED[:, 1] == v)[0].tolist() for v in range(_NVAR)]
_NUM_ITER = 20

_SQRT2 = 1.4142135
_LN2_HI = 0.69314575
_LN2_LO = 1.4286068e-06
_INVLN2 = 1.4426950408889634
_LOGC = [-6.900793061981325e-10, 0.9999999966211686, -0.4999996532905042,
         0.3333335964540943, -0.25002812462237173, 0.2000147231976227,
         -0.16586843259976403, 0.14176370184103376, -0.13388417444788703,
         0.12990627398860688, -0.07417228391986355]
_EXPC = [0.9999999999595321, 1.000000037739721, 0.5000000107781664,
         0.16666415422747397, 0.04166621818498068, 0.008375133426734613,
         0.0013948586767683234]
_RCPC = [2.8499173034659373, -2.9844526530961426, 1.3616308933192882,
         -0.22857251654217783]


def _horner(coefs, x):
    acc = jnp.full(x.shape, coefs[-1], x.dtype)
    for c in coefs[-2::-1]:
        acc = acc * x + c
    return acc


_RLO = float(1e-7 / (2.0 - 1e-7))
_RHI = float((2.0 - 1e-7) / 1e-7)


def _sc_log(e):
    i = plsc.bitcast(e, jnp.int32)
    k = (i >> 23) - 127
    m = plsc.bitcast((i & 0x7FFFFF) | 0x3F800000, jnp.float32)
    big = m > _SQRT2
    m = jnp.where(big, m * 0.5, m)
    kf = (k + jnp.where(big, 1, 0)).astype(jnp.float32)
    p = _horner(_LOGC, m - 1.0)
    return kf * _LN2_HI + (kf * _LN2_LO + p)


def _sc_atanh2(x):
    r = jnp.clip((1.0 + x) / (1.0 - x), _RLO, _RHI)
    return _sc_log(r)


def _sc_tanh12(y):
    t = jnp.exp(-jnp.abs(y))
    q = (1.0 - t) / (1.0 + t)
    return jnp.where(y < 0.0, -q, q)


def _loo_products(grp):
    n = len(grp)
    pref = [grp[0]]
    for k in range(1, n):
        pref.append(pref[-1] * grp[k])
    suf = [grp[n - 1]]
    for k in range(n - 2, -1, -1):
        suf.append(suf[-1] * grp[k])
    suf = suf[::-1]
    out = []
    for k in range(n):
        if k == 0:
            out.append(suf[1])
        elif k == n - 1:
            out.append(pref[n - 2])
        else:
            out.append(pref[k - 1] * suf[k + 1])
    return out


def _sc_bp_body(pb, unroll, llr_hbm, out_hbm, llr_v, msg_v, he_v, out_v):
    wid = lax.axis_index("s") * 2 + lax.axis_index("c")
    pltpu.sync_copy(llr_hbm.at[wid], llr_v)
    ng = pb // 16

    @plsc.parallel_loop(0, ng, unroll=unroll)
    def _init(g):
        lanes = pl.ds(g * 16, 16)
        for v in range(_NVAR):
            t = _sc_tanh12(llr_v[v, lanes])
            for e in _COL_EDGES[v]:
                msg_v[e, lanes] = t

    def it_body(it, c2):
        @plsc.parallel_loop(0, ng, unroll=unroll)
        def _g(g):
            lanes = pl.ds(g * 16, 16)
            cs = [None] * _NVAR
            for c in range(_NCHK):
                grp = [msg_v[c * _DEG + k, lanes] for k in range(_DEG)]
                loo = _loo_products(grp)
                for k in range(_DEG):
                    he = _sc_atanh2(loo[k])
                    e = c * _DEG + k
                    he_v[e, lanes] = he
                    v = int(_COLS[c, k])
                    cs[v] = he if cs[v] is None else cs[v] + he
            for v in range(_NVAR):
                base = cs[v] + llr_v[v, lanes]
                out_v[v, lanes] = base
                for e in _COL_EDGES[v]:
                    msg_v[e, lanes] = _sc_tanh12(base - he_v[e, lanes])
        return c2

    lax.fori_loop(0, _NUM_ITER, it_body, 0)
    pltpu.sync_copy(out_v, out_hbm.at[wid])


_NW = 32


@functools.partial(jax.jit, static_argnames=("pb", "unroll"))
def _sc_bp(llr_sc, pb, unroll=2):
    mesh = plsc.VectorSubcoreMesh(
        core_axis_name="c", subcore_axis_name="s", num_cores=2, num_subcores=16)
    return pl.kernel(
        functools.partial(_sc_bp_body, pb, unroll),
        out_type=jax.ShapeDtypeStruct((_NW, _NVAR, pb), jnp.float32),
        mesh=mesh,
        compiler_params=pltpu.CompilerParams(needs_layout_passes=False),
        scratch_types=[
            pltpu.VMEM((_NVAR, pb), jnp.float32),
            pltpu.VMEM((_E, pb), jnp.float32),
            pltpu.VMEM((_E, pb), jnp.float32),
            pltpu.VMEM((_NVAR, pb), jnp.float32),
        ],
    )(llr_sc)


def _tc_bp_block(llr_rows, s):
    h_r = [llr_rows[int(_COLS[c, k])] for c in range(_NCHK) for k in range(_DEG)]

    def body(_, carry):
        m_stack, _cs = carry
        msg = [m_stack[e * s:(e + 1) * s] for e in range(_E)]
        h_e = [None] * _E
        cs = [None] * _NVAR
        for c in range(_NCHK):
            loo = _loo_products(msg[c * _DEG:(c + 1) * _DEG])
            for k in range(_DEG):
                e1 = jnp.clip(1.0 + loo[k], 1e-07, 2.0 - 1e-07)
                e2 = jnp.clip(1.0 - loo[k], 1e-07, 2.0 - 1e-07)
                he = jnp.log(e1 / e2)
                e = c * _DEG + k
                h_e[e] = he
                v = int(_COLS[c, k])
                cs[v] = he if cs[v] is None else cs[v] + he
        new_msg = [
            jnp.tanh((cs[int(_COLS[c, k])] - h_e[c * _DEG + k]
                      + h_r[c * _DEG + k]) * 0.5)
            for c in range(_NCHK) for k in range(_DEG)
        ]
        return jnp.concatenate(new_msg, axis=0), jnp.concatenate(cs, axis=0)

    msg0 = [jnp.tanh(h * 0.5) for h in h_r]
    cs0 = jnp.zeros((_NVAR * s, llr_rows[0].shape[1]), jnp.float32)
    _, cs_fin = jax.lax.fori_loop(
        0, _NUM_ITER, body, (jnp.concatenate(msg0, axis=0), cs0))
    out = [cs_fin[v * s:(v + 1) * s] + llr_rows[v] for v in range(_NVAR)]
    return jnp.concatenate(out, axis=0)


def _tc_kernel_body(llr_ref, out_ref, *, s):
    llr_rows = [llr_ref[v * s:(v + 1) * s] for v in range(_NVAR)]
    out_ref[...] = _tc_bp_block(llr_rows, s)


def _tc_bp(llr_part, grid):
    Bt = llr_part.shape[0]
    S = 8
    W = Bt // S
    WT = W // grid
    llr2 = llr_part.T.reshape(_NVAR * S, W)
    out2 = pl.pallas_call(
        functools.partial(_tc_kernel_body, s=S),
        grid=(grid,),
        in_specs=[pl.BlockSpec((_NVAR * S, WT), lambda i: (0, i))],
        out_specs=pl.BlockSpec((_NVAR * S, WT), lambda i: (0, i)),
        out_shape=jax.ShapeDtypeStruct((_NVAR * S, W), jnp.float32),
    )(llr2)
    return out2.reshape(_NVAR, Bt).T


_B_SC = 512


@jax.jit
def kernel(llr):
    B = llr.shape[0]
    pb = _B_SC // _NW
    llr_sc = llr[:_B_SC].T.reshape(_NVAR, _NW, pb).transpose(1, 0, 2)
    out_sc = _sc_bp(llr_sc, pb, unroll=1)
    out_sc = out_sc.transpose(1, 0, 2).reshape(_NVAR, _B_SC).T
    llr_tc = jnp.concatenate(
        [llr[_B_SC:], jnp.zeros((_B_SC, _NVAR), jnp.float32)], axis=0)
    out_tc = _tc_bp(llr_tc, grid=4)[:B - _B_SC]
    return jnp.concatenate([out_sc, out_tc], axis=0)
